# baseline (speedup 1.0000x reference)
import functools

import jax
import jax.numpy as jnp
from jax import lax
from jax.experimental import pallas as pl
from jax.experimental.pallas import tpu as pltpu
from jax.experimental.pallas import tpu_sc as plsc

N = 10000
E = 320000
D_IN = 128
D_HID = 128
D_OUT = 64

NC = 2
NS = 16
NW = NC * NS
EPW = E // NW
B = 80
CHUNKS = EPW // B
NPAD = 10240
RPT = NPAD // NS

_MESH = plsc.VectorSubcoreMesh(core_axis_name="c", subcore_axis_name="s")


SUP = 25
NSUP = CHUNKS // SUP


@functools.partial(
    pl.kernel,
    out_type=jax.ShapeDtypeStruct((NC, NPAD, 128), jnp.float32),
    mesh=_MESH,
    scratch_types=[
        pltpu.VMEM((B,), jnp.int32),
        pltpu.VMEM((B, 128), jnp.float32),
        pltpu.VMEM_SHARED((NPAD, 128), jnp.float32),
    ],
)
def _deg_kernel(dst_hbm, ones_hbm, zeros_hbm, out_hbm, didx, ones_v, acc):
    c = lax.axis_index("c")
    s = lax.axis_index("s")
    w = c * NS + s
    r0 = s * RPT
    pltpu.sync_copy(zeros_hbm.at[pl.ds(r0, RPT)], acc.at[pl.ds(r0, RPT)])
    pltpu.sync_copy(ones_hbm, ones_v)
    plsc.subcore_barrier()

    def body(i, carry):
        base = pl.multiple_of(w * EPW + i * B, B)
        pltpu.sync_copy(dst_hbm.at[pl.ds(base, B)], didx)
        pltpu.sync_copy(ones_v, acc.at[didx], add=True)
        return carry

    lax.fori_loop(0, CHUNKS, body, 0)
    plsc.subcore_barrier()
    pltpu.sync_copy(acc.at[pl.ds(r0, RPT)], out_hbm.at[c, pl.ds(r0, RPT)])


def _make_agg(D):
    @functools.partial(
        pl.kernel,
        out_type=jax.ShapeDtypeStruct((NC, NPAD, D), jnp.float32),
        mesh=_MESH,
        scratch_types=[
            pltpu.VMEM((SUP, B), jnp.int32),
            pltpu.VMEM((SUP, B), jnp.int32),
            pltpu.VMEM((B, D), jnp.float32),
            pltpu.VMEM((B, D), jnp.float32),
            pltpu.VMEM_SHARED((NPAD, D), jnp.float32),
            pltpu.SemaphoreType.DMA,
            pltpu.SemaphoreType.DMA,
            pltpu.SemaphoreType.DMA,
            pltpu.SemaphoreType.DMA,
        ],
    )
    def agg(y_hbm, src4_hbm, dst4_hbm, zeros_hbm, out_hbm,
            sidx, didx, rows_a, rows_b, acc, g_a, g_b, s_a, s_b):
        c = lax.axis_index("c")
        s = lax.axis_index("s")
        w = c * NS + s
        r0 = s * RPT
        pltpu.sync_copy(zeros_hbm.at[pl.ds(r0, RPT)], acc.at[pl.ds(r0, RPT)])
        plsc.subcore_barrier()

        def fire_gather(i, rows, sem):
            pltpu.async_copy(y_hbm.at[sidx.at[i]], rows, sem)

        def drain_gather(rows, sem):
            pltpu.make_async_copy(y_hbm.at[pl.ds(0, B)], rows, sem).wait()

        def fire_scatter(i, rows, sem):
            pltpu.async_copy(rows, acc.at[didx.at[i]], sem, add=True)

        def drain_scatter(rows, sem):
            pltpu.make_async_copy(rows, acc.at[didx.at[0]], sem).wait()

        def super_body(u, carry):
            pltpu.sync_copy(src4_hbm.at[w, u], sidx)
            pltpu.sync_copy(dst4_hbm.at[w, u], didx)
            fire_gather(0, rows_a, g_a)

            def body(k, carry2):
                i0 = 2 * k
                drain_gather(rows_a, g_a)
                fire_gather(i0 + 1, rows_b, g_b)
                fire_scatter(i0, rows_a, s_a)
                drain_scatter(rows_a, s_a)
                fire_gather(i0 + 2, rows_a, g_a)
                drain_gather(rows_b, g_b)
                fire_scatter(i0 + 1, rows_b, s_b)
                drain_scatter(rows_b, s_b)
                return carry2

            lax.fori_loop(0, (SUP - 1) // 2, body, 0)
            drain_gather(rows_a, g_a)
            fire_scatter(SUP - 1, rows_a, s_a)
            drain_scatter(rows_a, s_a)
            return carry

        lax.fori_loop(0, NSUP, super_body, 0)
        plsc.subcore_barrier()
        pltpu.sync_copy(acc.at[pl.ds(r0, RPT)], out_hbm.at[c, pl.ds(r0, RPT)])

    return agg


_agg128 = _make_agg(D_HID)


_ROWS = 400
_GRID = N // _ROWS


def _tc1_body(x_ref, w1_ref, degp_ref, y1_ref, dinv_ref):
    deg = degp_ref[0] + degp_ref[1] + 1.0
    dinv = lax.rsqrt(deg)
    dinv_ref[...] = dinv[:, 0:16]
    xw = jnp.dot(x_ref[...], w1_ref[...], preferred_element_type=jnp.float32)
    y1_ref[...] = xw * dinv


def _tc2_body(p_ref, y1_ref, dinv_ref, b1_ref, w2_ref, y2_ref):
    dinv = dinv_ref[:, 0:1]
    h = (p_ref[0] + p_ref[1] + y1_ref[...]) * dinv + b1_ref[...]
    h = jnp.maximum(h, 0.0)
    y2_ref[...] = jnp.dot(h, w2_ref[...], preferred_element_type=jnp.float32) * dinv


def _tc3_body(p_ref, y2_ref, dinv_ref, b2_ref, out_ref):
    dinv = dinv_ref[:, 0:1]
    acc = p_ref[0, :, :D_OUT] + p_ref[1, :, :D_OUT] + y2_ref[:, :D_OUT]
    out_ref[...] = acc * dinv + b2_ref[...]


def _row_spec(d):
    return pl.BlockSpec((_ROWS, d), lambda i: (i, 0))


def _part_spec(d):
    return pl.BlockSpec((NC, _ROWS, d), lambda i: (0, i, 0))


def _full_spec(r, d):
    return pl.BlockSpec((r, d), lambda i: (0, 0))


def kernel(x, edge_index, W1, b1, W2, b2):
    src = edge_index[0].astype(jnp.int32).reshape(NW, NSUP, SUP, B)
    dst = edge_index[1].astype(jnp.int32).reshape(NW, NSUP, SUP, B)
    dst1 = edge_index[1].astype(jnp.int32)
    ones128 = jnp.ones((B, 128), jnp.float32)
    zeros128 = jnp.zeros((NPAD, D_HID), jnp.float32)
    W2p = jnp.concatenate([W2, jnp.zeros((D_HID, D_HID - D_OUT), jnp.float32)], axis=1)
    b1r = b1.reshape(1, D_HID)
    b2r = b2.reshape(1, D_OUT)

    degp = _deg_kernel(dst1, ones128, zeros128)

    y1, dinv = pl.pallas_call(
        _tc1_body,
        grid=(_GRID,),
        in_specs=[_row_spec(D_IN), _full_spec(D_IN, D_HID), _part_spec(128)],
        out_specs=[_row_spec(D_HID), _row_spec(16)],
        out_shape=[
            jax.ShapeDtypeStruct((N, D_HID), jnp.float32),
            jax.ShapeDtypeStruct((N, 16), jnp.float32),
        ],
    )(x, W1, degp)

    p1 = _agg128(y1, src, dst, zeros128)

    y2 = pl.pallas_call(
        _tc2_body,
        grid=(_GRID,),
        in_specs=[_part_spec(D_HID), _row_spec(D_HID), _row_spec(16),
                  _full_spec(1, D_HID), _full_spec(D_HID, D_HID)],
        out_specs=_row_spec(D_HID),
        out_shape=jax.ShapeDtypeStruct((N, D_HID), jnp.float32),
    )(p1, y1, dinv, b1r, W2p)

    p2 = _agg128(y2, src, dst, zeros128)

    out = pl.pallas_call(
        _tc3_body,
        grid=(_GRID,),
        in_specs=[_part_spec(D_HID), _row_spec(D_HID), _row_spec(16),
                  _full_spec(1, D_OUT)],
        out_specs=_row_spec(D_OUT),
        out_shape=jax.ShapeDtypeStruct((N, D_OUT), jnp.float32),
    )(p2, y2, dinv, b2r)

    return out


# speedup vs baseline: 1.3399x; 1.3399x over previous
import functools

import jax
import jax.numpy as jnp
from jax import lax
from jax.experimental import pallas as pl
from jax.experimental.pallas import tpu as pltpu
from jax.experimental.pallas import tpu_sc as plsc

N = 10000
E = 320000
D_IN = 128
D_HID = 128
D_OUT = 64

NC = 2
NS = 16
NW = NC * NS
EPW = E // NW
B = 80
CHUNKS = EPW // B
NPAD = 10240
RPT = NPAD // NS

_MESH = plsc.VectorSubcoreMesh(core_axis_name="c", subcore_axis_name="s")


SUP = 25
NSUP = CHUNKS // SUP


@functools.partial(
    pl.kernel,
    out_type=jax.ShapeDtypeStruct((NW, NPAD), jnp.float32),
    mesh=_MESH,
    scratch_types=[
        pltpu.VMEM((EPW,), jnp.int32),
        pltpu.VMEM((NPAD,), jnp.float32),
    ],
    compiler_params=pltpu.CompilerParams(needs_layout_passes=False),
)
def _deg_kernel(dst_hbm, zeros_hbm, out_hbm, didx, hist):
    c = lax.axis_index("c")
    s = lax.axis_index("s")
    w = c * NS + s
    base = pl.multiple_of(w * EPW, EPW)
    pltpu.sync_copy(dst_hbm.at[pl.ds(base, EPW)], didx)
    pltpu.sync_copy(zeros_hbm, hist)
    ones = jnp.ones((16,), jnp.float32)

    def body(j, carry):
        idx = didx[pl.ds(pl.multiple_of(j * 16, 16), 16)]
        plsc.addupdate_scatter(hist, [idx], ones)
        return carry

    lax.fori_loop(0, EPW // 16, body, 0)
    pltpu.sync_copy(hist, out_hbm.at[w])


def _make_agg(D):
    @functools.partial(
        pl.kernel,
        out_type=jax.ShapeDtypeStruct((NC, NPAD, D), jnp.float32),
        mesh=_MESH,
        scratch_types=[
            pltpu.VMEM((SUP, B), jnp.int32),
            pltpu.VMEM((SUP, B), jnp.int32),
            pltpu.VMEM((B, D), jnp.float32),
            pltpu.VMEM((B, D), jnp.float32),
            pltpu.VMEM_SHARED((NPAD, D), jnp.float32),
            pltpu.SemaphoreType.DMA,
            pltpu.SemaphoreType.DMA,
            pltpu.SemaphoreType.DMA,
            pltpu.SemaphoreType.DMA,
        ],
    )
    def agg(y_hbm, src4_hbm, dst4_hbm, zeros_hbm, out_hbm,
            sidx, didx, rows_a, rows_b, acc, g_a, g_b, s_a, s_b):
        c = lax.axis_index("c")
        s = lax.axis_index("s")
        w = c * NS + s
        r0 = s * RPT
        pltpu.sync_copy(zeros_hbm.at[pl.ds(r0, RPT)], acc.at[pl.ds(r0, RPT)])
        plsc.subcore_barrier()

        def fire_gather(i, rows, sem):
            pltpu.async_copy(y_hbm.at[sidx.at[i]], rows, sem)

        def drain_gather(rows, sem):
            pltpu.make_async_copy(y_hbm.at[pl.ds(0, B)], rows, sem).wait()

        def fire_scatter(i, rows, sem):
            pltpu.async_copy(rows, acc.at[didx.at[i]], sem, add=True)

        def drain_scatter(rows, sem):
            pltpu.make_async_copy(rows, acc.at[didx.at[0]], sem).wait()

        def super_body(u, carry):
            pltpu.sync_copy(src4_hbm.at[w, u], sidx)
            pltpu.sync_copy(dst4_hbm.at[w, u], didx)
            fire_gather(0, rows_a, g_a)

            def body(k, carry2):
                i0 = 2 * k
                drain_gather(rows_a, g_a)
                fire_gather(i0 + 1, rows_b, g_b)
                fire_scatter(i0, rows_a, s_a)
                drain_scatter(rows_a, s_a)
                fire_gather(i0 + 2, rows_a, g_a)
                drain_gather(rows_b, g_b)
                fire_scatter(i0 + 1, rows_b, s_b)
                drain_scatter(rows_b, s_b)
                return carry2

            lax.fori_loop(0, (SUP - 1) // 2, body, 0)
            drain_gather(rows_a, g_a)
            fire_scatter(SUP - 1, rows_a, s_a)
            drain_scatter(rows_a, s_a)
            return carry

        lax.fori_loop(0, NSUP, super_body, 0)
        plsc.subcore_barrier()
        pltpu.sync_copy(acc.at[pl.ds(r0, RPT)], out_hbm.at[c, pl.ds(r0, RPT)])

    return agg


_agg128 = _make_agg(D_HID)


_ROWS = 512
_GRID = (N + _ROWS - 1) // _ROWS


def _tc1_body(x_ref, w1_ref, degp_ref, y1_ref, dinv_ref):
    deg = jnp.sum(degp_ref[...], axis=0) + 1.0
    dinv = lax.rsqrt(deg)[:, None]
    dinv_ref[...] = jnp.broadcast_to(dinv, (_ROWS, 16))
    xw = jnp.dot(x_ref[...], w1_ref[...], preferred_element_type=jnp.float32)
    y1_ref[...] = xw * dinv


def _tc2_body(p_ref, y1_ref, dinv_ref, b1_ref, w2_ref, y2_ref):
    dinv = dinv_ref[:, 0:1]
    h = (p_ref[0] + p_ref[1] + y1_ref[...]) * dinv + b1_ref[...]
    h = jnp.maximum(h, 0.0)
    y2_ref[...] = jnp.dot(h, w2_ref[...], preferred_element_type=jnp.float32) * dinv


def _tc3_body(p_ref, y2_ref, dinv_ref, b2_ref, out_ref):
    dinv = dinv_ref[:, 0:1]
    acc = p_ref[0, :, :D_OUT] + p_ref[1, :, :D_OUT] + y2_ref[:, :D_OUT]
    out_ref[...] = acc * dinv + b2_ref[...]


def _row_spec(d):
    return pl.BlockSpec((_ROWS, d), lambda i: (i, 0))


def _part_spec(d):
    return pl.BlockSpec((NC, _ROWS, d), lambda i: (0, i, 0))


def _full_spec(r, d):
    return pl.BlockSpec((r, d), lambda i: (0, 0))


def kernel(x, edge_index, W1, b1, W2, b2):
    src = edge_index[0].astype(jnp.int32).reshape(NW, NSUP, SUP, B)
    dst = edge_index[1].astype(jnp.int32).reshape(NW, NSUP, SUP, B)
    dst1 = edge_index[1].astype(jnp.int32)
    zeros1 = jnp.zeros((NPAD,), jnp.float32)
    zeros128 = jnp.zeros((NPAD, D_HID), jnp.float32)
    W2p = jnp.concatenate([W2, jnp.zeros((D_HID, D_HID - D_OUT), jnp.float32)], axis=1)
    b1r = b1.reshape(1, D_HID)
    b2r = b2.reshape(1, D_OUT)

    degp = _deg_kernel(dst1, zeros1)

    y1, dinv = pl.pallas_call(
        _tc1_body,
        grid=(_GRID,),
        in_specs=[_row_spec(D_IN), _full_spec(D_IN, D_HID),
                  pl.BlockSpec((NW, _ROWS), lambda i: (0, i))],
        out_specs=[_row_spec(D_HID), _row_spec(16)],
        out_shape=[
            jax.ShapeDtypeStruct((N, D_HID), jnp.float32),
            jax.ShapeDtypeStruct((N, 16), jnp.float32),
        ],
    )(x, W1, degp)

    p1 = _agg128(y1, src, dst, zeros128)

    y2 = pl.pallas_call(
        _tc2_body,
        grid=(_GRID,),
        in_specs=[_part_spec(D_HID), _row_spec(D_HID), _row_spec(16),
                  _full_spec(1, D_HID), _full_spec(D_HID, D_HID)],
        out_specs=_row_spec(D_HID),
        out_shape=jax.ShapeDtypeStruct((N, D_HID), jnp.float32),
    )(p1, y1, dinv, b1r, W2p)

    p2 = _agg128(y2, src, dst, zeros128)

    out = pl.pallas_call(
        _tc3_body,
        grid=(_GRID,),
        in_specs=[_part_spec(D_HID), _row_spec(D_HID), _row_spec(16),
                  _full_spec(1, D_OUT)],
        out_specs=_row_spec(D_OUT),
        out_shape=jax.ShapeDtypeStruct((N, D_OUT), jnp.float32),
    )(p2, y2, dinv, b2r)

    return out
